# baseline (speedup 1.0000x reference)
import jax
import jax.numpy as jnp
from jax import lax
from jax.experimental import pallas as pl
from jax.experimental.pallas import tpu as pltpu
from jax.experimental.pallas import tpu_sc as plsc

T = 8192
D = 2048
E = 16
TOP_K = 2

NHALF = 2
TH = T // NHALF

CHUNK = 512
NCHUNK = TH // CHUNK
NBUF = 4

NC = 2
NS = 16
NW = NC * NS
TPW = TH // NW
LANES = 16


def _make_matmul_body(h):
    row0 = h * TH

    def body(x_hbm, w_ref, o_ref, xbuf, sems):
        def issue(c, buf):
            pltpu.make_async_copy(
                x_hbm.at[pl.ds(row0 + c * CHUNK, CHUNK), :],
                xbuf.at[buf], sems.at[buf],
            ).start()

        for c in range(NBUF):
            issue(c, c)

        def outer(o, carry):
            for b in range(NBUF):
                c = o * NBUF + b
                pltpu.make_async_copy(
                    x_hbm.at[pl.ds(row0 + c * CHUNK, CHUNK), :],
                    xbuf.at[b], sems.at[b],
                ).wait()
                st = lax.dot_general(
                    w_ref[...], xbuf[b],
                    (((1,), (1,)), ((), ())),
                    preferred_element_type=jnp.float32,
                )
                for i in range(CHUNK // TPW):
                    o_ref[pl.ds(c * (CHUNK // TPW) + i, 1)] = (
                        st[:, i * TPW:(i + 1) * TPW].reshape(1, E, TPW))

                @pl.when(c + NBUF < NCHUNK)
                def _():
                    issue(c + NBUF, b)

            return carry

        lax.fori_loop(0, NCHUNK // NBUF, outer, 0, unroll=False)

    return body


def _gate_matmul(x, w, h):
    return pl.pallas_call(
        _make_matmul_body(h),
        in_specs=[
            pl.BlockSpec(memory_space=pl.ANY),
            pl.BlockSpec(memory_space=pltpu.VMEM),
        ],
        out_specs=pl.BlockSpec(memory_space=pltpu.VMEM),
        out_shape=jax.ShapeDtypeStruct((NW, E, TPW), jnp.float32),
        compiler_params=pltpu.CompilerParams(skip_device_barrier=True),
        scratch_shapes=[
            pltpu.VMEM((NBUF, CHUNK, D), jnp.float32),
            pltpu.SemaphoreType.DMA((NBUF,)),
        ],
    )(x, w)


def _router_body(scores_hbm, b_hbm, oi1, oi2, ov1, ov2,
                 s_v, b_v, i1_v, i2_v, v1_v, v2_v):
    wid = lax.axis_index("s") * NC + lax.axis_index("c")
    base = wid * TPW
    pltpu.sync_copy(scores_hbm.at[wid], s_v)
    pltpu.sync_copy(b_hbm, b_v)

    b_full = b_v[...]
    b_sc = [b_full[e] for e in range(E)]
    neg_inf = jnp.full((LANES,), -jnp.inf, jnp.float32)

    def group(g, carry):
        sl = pl.ds(g * LANES, LANES)
        s_list = [s_v[e, sl] + b_sc[e] for e in range(E)]

        m1 = s_list[0]
        i1 = jnp.zeros((LANES,), jnp.int32)
        for e in range(1, E):
            gt = s_list[e] > m1
            i1 = jnp.where(gt, jnp.full((LANES,), e, jnp.int32), i1)
            m1 = jnp.where(gt, s_list[e], m1)
        m2 = neg_inf
        i2 = jnp.zeros((LANES,), jnp.int32)
        for e in range(E):
            cand = jnp.where(i1 == e, neg_inf, s_list[e])
            gt = cand > m2
            i2 = jnp.where(gt, jnp.full((LANES,), e, jnp.int32), i2)
            m2 = jnp.where(gt, cand, m2)

        sumexp = jnp.exp(s_list[0] - m1)
        for e in range(1, E):
            sumexp = sumexp + jnp.exp(s_list[e] - m1)
        i1_v[sl] = i1
        i2_v[sl] = i2
        v1_v[sl] = jnp.full((LANES,), 1.0, jnp.float32) / sumexp
        v2_v[sl] = jnp.exp(m2 - m1) / sumexp
        return carry

    lax.fori_loop(0, TPW // LANES, group, 0, unroll=False)

    out_sl = pl.ds(base, TPW)
    pltpu.sync_copy(i1_v, oi1.at[out_sl])
    pltpu.sync_copy(i2_v, oi2.at[out_sl])
    pltpu.sync_copy(v1_v, ov1.at[out_sl])
    pltpu.sync_copy(v2_v, ov2.at[out_sl])


def _router(scores, b):
    run = pl.kernel(
        _router_body,
        out_type=[
            jax.ShapeDtypeStruct((TH,), jnp.int32),
            jax.ShapeDtypeStruct((TH,), jnp.int32),
            jax.ShapeDtypeStruct((TH,), jnp.float32),
            jax.ShapeDtypeStruct((TH,), jnp.float32),
        ],
        mesh=plsc.VectorSubcoreMesh(core_axis_name="c", subcore_axis_name="s"),
        compiler_params=pltpu.CompilerParams(
            needs_layout_passes=False,
            skip_device_barrier=True,
        ),
        scratch_types=[
            pltpu.VMEM((E, TPW), jnp.float32),
            pltpu.VMEM((E,), jnp.float32),
            pltpu.VMEM((TPW,), jnp.int32),
            pltpu.VMEM((TPW,), jnp.int32),
            pltpu.VMEM((TPW,), jnp.float32),
            pltpu.VMEM((TPW,), jnp.float32),
        ],
    )
    return run(scores, b)


def kernel(x, W, b):
    halves = []
    for h in range(NHALF):
        scores_h = _gate_matmul(x, W, h)
        halves.append(_router(scores_h, b))
    i1 = jnp.concatenate([halves[0][0], halves[1][0]])
    i2 = jnp.concatenate([halves[0][1], halves[1][1]])
    v1 = jnp.concatenate([halves[0][2], halves[1][2]])
    v2 = jnp.concatenate([halves[0][3], halves[1][3]])
    idx = jnp.stack([i1, i2], axis=1)
    val = jnp.stack([v1, v2], axis=1)
    return (idx, val)


# speedup vs baseline: 1.0642x; 1.0642x over previous
import jax
import jax.numpy as jnp
from jax import lax
from jax.experimental import pallas as pl
from jax.experimental.pallas import tpu as pltpu
from jax.experimental.pallas import tpu_sc as plsc

T = 8192
D = 2048
E = 16
TOP_K = 2

CHUNK = 512
NCHUNK = T // CHUNK
NBUF = 4

NC = 2
NS = 16
NW = NC * NS
TPW = T // NW
LANES = 16


def _gate_matmul_body(x_hbm, w_ref, o_ref, xbuf, sems):
    def issue(c, buf):
        pltpu.make_async_copy(
            x_hbm.at[pl.ds(c * CHUNK, CHUNK), :], xbuf.at[buf], sems.at[buf]
        ).start()

    for c in range(NBUF):
        issue(c, c)

    def outer(o, carry):
        for b in range(NBUF):
            c = o * NBUF + b
            pltpu.make_async_copy(
                x_hbm.at[pl.ds(c * CHUNK, CHUNK), :], xbuf.at[b], sems.at[b]
            ).wait()
            st = lax.dot_general(
                w_ref[...], xbuf[b],
                (((1,), (1,)), ((), ())),
                preferred_element_type=jnp.float32,
            )
            for i in range(CHUNK // TPW):
                o_ref[pl.ds(c * (CHUNK // TPW) + i, 1)] = (
                    st[:, i * TPW:(i + 1) * TPW].reshape(1, E, TPW))

            @pl.when(c + NBUF < NCHUNK)
            def _():
                issue(c + NBUF, b)

        return carry

    lax.fori_loop(0, NCHUNK // NBUF, outer, 0, unroll=False)


def _gate_matmul(x, w):
    return pl.pallas_call(
        _gate_matmul_body,
        in_specs=[
            pl.BlockSpec(memory_space=pl.ANY),
            pl.BlockSpec(memory_space=pltpu.VMEM),
        ],
        out_specs=pl.BlockSpec(memory_space=pltpu.VMEM),
        out_shape=jax.ShapeDtypeStruct((NW, E, TPW), jnp.float32),
        compiler_params=pltpu.CompilerParams(skip_device_barrier=True),
        scratch_shapes=[
            pltpu.VMEM((NBUF, CHUNK, D), jnp.float32),
            pltpu.SemaphoreType.DMA((NBUF,)),
        ],
    )(x, w)


def _router_body(scores_hbm, b_hbm, oi1, oi2, ov1, ov2,
                 s_v, b_v, i1_v, i2_v, v1_v, v2_v):
    wid = lax.axis_index("s") * NC + lax.axis_index("c")
    base = wid * TPW
    pltpu.sync_copy(scores_hbm.at[wid], s_v)
    pltpu.sync_copy(b_hbm, b_v)

    b_full = b_v[...]
    b_sc = [b_full[e] for e in range(E)]
    neg_inf = jnp.full((LANES,), -jnp.inf, jnp.float32)

    def group(g, carry):
        sl = pl.ds(g * LANES, LANES)
        s_list = [s_v[e, sl] + b_sc[e] for e in range(E)]

        m1 = s_list[0]
        i1 = jnp.zeros((LANES,), jnp.int32)
        for e in range(1, E):
            gt = s_list[e] > m1
            i1 = jnp.where(gt, jnp.full((LANES,), e, jnp.int32), i1)
            m1 = jnp.where(gt, s_list[e], m1)
        m2 = neg_inf
        i2 = jnp.zeros((LANES,), jnp.int32)
        for e in range(E):
            cand = jnp.where(i1 == e, neg_inf, s_list[e])
            gt = cand > m2
            i2 = jnp.where(gt, jnp.full((LANES,), e, jnp.int32), i2)
            m2 = jnp.where(gt, cand, m2)

        sumexp = jnp.exp(s_list[0] - m1)
        for e in range(1, E):
            sumexp = sumexp + jnp.exp(s_list[e] - m1)
        i1_v[sl] = i1
        i2_v[sl] = i2
        v1_v[sl] = jnp.full((LANES,), 1.0, jnp.float32) / sumexp
        v2_v[sl] = jnp.exp(m2 - m1) / sumexp
        return carry

    lax.fori_loop(0, TPW // LANES, group, 0, unroll=False)

    out_sl = pl.ds(base, TPW)
    pltpu.sync_copy(i1_v, oi1.at[out_sl])
    pltpu.sync_copy(i2_v, oi2.at[out_sl])
    pltpu.sync_copy(v1_v, ov1.at[out_sl])
    pltpu.sync_copy(v2_v, ov2.at[out_sl])


def _router(scores, b):
    run = pl.kernel(
        _router_body,
        out_type=[
            jax.ShapeDtypeStruct((T,), jnp.int32),
            jax.ShapeDtypeStruct((T,), jnp.int32),
            jax.ShapeDtypeStruct((T,), jnp.float32),
            jax.ShapeDtypeStruct((T,), jnp.float32),
        ],
        mesh=plsc.VectorSubcoreMesh(core_axis_name="c", subcore_axis_name="s"),
        compiler_params=pltpu.CompilerParams(
            needs_layout_passes=False,
            skip_device_barrier=True,
        ),
        scratch_types=[
            pltpu.VMEM((E, TPW), jnp.float32),
            pltpu.VMEM((E,), jnp.float32),
            pltpu.VMEM((TPW,), jnp.int32),
            pltpu.VMEM((TPW,), jnp.int32),
            pltpu.VMEM((TPW,), jnp.float32),
            pltpu.VMEM((TPW,), jnp.float32),
        ],
    )
    return run(scores, b)


def kernel(x, W, b):
    scores_t = _gate_matmul(x, W)
    i1, i2, v1, v2 = _router(scores_t, b)
    idx = jnp.stack([i1, i2], axis=1)
    val = jnp.stack([v1, v2], axis=1)
    return (idx, val)


# speedup vs baseline: 1.0781x; 1.0131x over previous
import jax
import jax.numpy as jnp
from jax import lax
from jax.experimental import pallas as pl
from jax.experimental.pallas import tpu as pltpu
from jax.experimental.pallas import tpu_sc as plsc

T = 8192
D = 2048
E = 16
TOP_K = 2

CHUNK = 512
NCHUNK = T // CHUNK
NBUF = 4

NC = 2
NS = 16
NW = NC * NS
TPW = T // NW
LANES = 16


def _gate_matmul_body(x_hbm, w_ref, o_ref, xbuf, sems):
    def issue(c, buf):
        pltpu.make_async_copy(
            x_hbm.at[pl.ds(c * CHUNK, CHUNK), :], xbuf.at[buf], sems.at[buf]
        ).start()

    for c in range(NBUF):
        issue(c, c)

    def outer(o, carry):
        for b in range(NBUF):
            c = o * NBUF + b
            pltpu.make_async_copy(
                x_hbm.at[pl.ds(c * CHUNK, CHUNK), :], xbuf.at[b], sems.at[b]
            ).wait()
            st = lax.dot_general(
                w_ref[...], xbuf[b],
                (((1,), (1,)), ((), ())),
                preferred_element_type=jnp.float32,
            )
            for i in range(CHUNK // TPW):
                o_ref[pl.ds(c * (CHUNK // TPW) + i, 1)] = (
                    st[:, i * TPW:(i + 1) * TPW].reshape(1, E, TPW))

            @pl.when(c + NBUF < NCHUNK)
            def _():
                issue(c + NBUF, b)

        return carry

    lax.fori_loop(0, NCHUNK // NBUF, outer, 0, unroll=False)


def _gate_matmul(x, w):
    return pl.pallas_call(
        _gate_matmul_body,
        in_specs=[
            pl.BlockSpec(memory_space=pl.ANY),
            pl.BlockSpec(memory_space=pltpu.VMEM),
        ],
        out_specs=pl.BlockSpec(memory_space=pltpu.VMEM),
        out_shape=jax.ShapeDtypeStruct((NW, E, TPW), jnp.float32),
        compiler_params=pltpu.CompilerParams(skip_device_barrier=True),
        scratch_shapes=[
            pltpu.VMEM((NBUF, CHUNK, D), jnp.float32),
            pltpu.SemaphoreType.DMA((NBUF,)),
        ],
    )(x, w)


def _router_body(scores_hbm, b_hbm, oi1, oi2, ov1, ov2,
                 s_v, b_v, i1_v, i2_v, v1_v, v2_v):
    wid = lax.axis_index("s") * NC + lax.axis_index("c")
    base = wid * TPW
    pltpu.sync_copy(scores_hbm.at[wid], s_v)
    pltpu.sync_copy(b_hbm, b_v)

    b_full = b_v[...]
    b_sc = [b_full[e] for e in range(E)]
    neg_inf = jnp.full((LANES,), -jnp.inf, jnp.float32)

    @plsc.parallel_loop(0, TPW // LANES, unroll=2)
    def group(g):
        sl = pl.ds(g * LANES, LANES)
        s_list = [s_v[e, sl] + b_sc[e] for e in range(E)]

        m1 = s_list[0]
        i1 = jnp.zeros((LANES,), jnp.int32)
        for e in range(1, E):
            gt = s_list[e] > m1
            i1 = jnp.where(gt, jnp.full((LANES,), e, jnp.int32), i1)
            m1 = jnp.where(gt, s_list[e], m1)
        m2 = neg_inf
        i2 = jnp.zeros((LANES,), jnp.int32)
        for e in range(E):
            cand = jnp.where(i1 == e, neg_inf, s_list[e])
            gt = cand > m2
            i2 = jnp.where(gt, jnp.full((LANES,), e, jnp.int32), i2)
            m2 = jnp.where(gt, cand, m2)

        sumexp = jnp.exp(s_list[0] - m1)
        for e in range(1, E):
            sumexp = sumexp + jnp.exp(s_list[e] - m1)
        i1_v[sl] = i1
        i2_v[sl] = i2
        v1_v[sl] = jnp.full((LANES,), 1.0, jnp.float32) / sumexp
        v2_v[sl] = jnp.exp(m2 - m1) / sumexp

    out_sl = pl.ds(base, TPW)
    pltpu.sync_copy(i1_v, oi1.at[out_sl])
    pltpu.sync_copy(i2_v, oi2.at[out_sl])
    pltpu.sync_copy(v1_v, ov1.at[out_sl])
    pltpu.sync_copy(v2_v, ov2.at[out_sl])


def _router(scores, b):
    run = pl.kernel(
        _router_body,
        out_type=[
            jax.ShapeDtypeStruct((T,), jnp.int32),
            jax.ShapeDtypeStruct((T,), jnp.int32),
            jax.ShapeDtypeStruct((T,), jnp.float32),
            jax.ShapeDtypeStruct((T,), jnp.float32),
        ],
        mesh=plsc.VectorSubcoreMesh(core_axis_name="c", subcore_axis_name="s"),
        compiler_params=pltpu.CompilerParams(
            needs_layout_passes=False,
            skip_device_barrier=True,
        ),
        scratch_types=[
            pltpu.VMEM((E, TPW), jnp.float32),
            pltpu.VMEM((E,), jnp.float32),
            pltpu.VMEM((TPW,), jnp.int32),
            pltpu.VMEM((TPW,), jnp.int32),
            pltpu.VMEM((TPW,), jnp.float32),
            pltpu.VMEM((TPW,), jnp.float32),
        ],
    )
    return run(scores, b)


def kernel(x, W, b):
    scores_t = _gate_matmul(x, W)
    i1, i2, v1, v2 = _router(scores_t, b)
    idx = jnp.stack([i1, i2], axis=1)
    val = jnp.stack([v1, v2], axis=1)
    return (idx, val)
